# baseline (speedup 1.0000x reference)
import functools

import jax
import jax.numpy as jnp
from jax import lax
from jax.experimental import pallas as pl
from jax.experimental.pallas import tpu as pltpu
from jax.experimental.pallas import tpu_sc as plsc

N = 10000
D = 128
E = 320000
NC = 2
NS = 16
NW = NC * NS
EPW = E // NW
CH = 128
HALF = N // 2
NPH = 5632
RPS = NPH // NS
ZB = 16
CAP = 10112
TRASHM = 511
NP2 = 10240
MSL = 512

_mesh = plsc.VectorSubcoreMesh(core_axis_name="c", subcore_axis_name="s",
                               num_cores=NC)


@functools.partial(
    pl.kernel,
    out_type=(
        jax.ShapeDtypeStruct((NW * NP2,), jnp.float32),
        jax.ShapeDtypeStruct((2 * NW * CAP,), jnp.int32),
        jax.ShapeDtypeStruct((2 * NW * CAP,), jnp.int32),
        jax.ShapeDtypeStruct((2 * NW * 16,), jnp.int32),
    ),
    mesh=_mesh,
    scratch_types=[
        pltpu.VMEM((EPW,), jnp.int32),
        pltpu.VMEM((EPW,), jnp.int32),
        pltpu.VMEM((NP2,), jnp.float32),
        pltpu.VMEM((CAP,), jnp.int32),
        pltpu.VMEM((CAP,), jnp.int32),
        pltpu.VMEM((CAP,), jnp.int32),
        pltpu.VMEM((CAP,), jnp.int32),
        pltpu.VMEM((16,), jnp.int32),
    ],
    compiler_params=pltpu.CompilerParams(needs_layout_passes=False),
)
def _route_kernel(src_hbm, dst_hbm, deg_hbm, rsrc_hbm, rdst_hbm, cnt_hbm,
                  src_v, dst_v, hist_v, os0_v, od0_v, os1_v, od1_v, cnt_v):
    cid = lax.axis_index("c")
    sid = lax.axis_index("s")
    wid = sid * NC + cid
    pltpu.sync_copy(src_hbm.at[pl.ds(wid * EPW, EPW)], src_v)
    pltpu.sync_copy(dst_hbm.at[pl.ds(wid * EPW, EPW)], dst_v)

    zeros16 = jnp.zeros((16,), jnp.float32)

    def zbody(i, carry):
        hist_v[pl.ds(i * 16, 16)] = zeros16
        return carry

    lax.fori_loop(0, NP2 // 16, zbody, 0)

    ones16 = jnp.ones((16,), jnp.float32)

    def hbody(i, carry):
        idx16 = dst_v[pl.ds(i * 16, 16)]
        plsc.addupdate_scatter(hist_v, [idx16], ones16)
        return carry

    lax.fori_loop(0, EPW // 16, hbody, 0)
    pltpu.sync_copy(hist_v, deg_hbm.at[pl.ds(wid * NP2, NP2)])

    def rbody(i, offs):
        off0, off1 = offs
        s16 = src_v[pl.ds(i * 16, 16)]
        d16 = dst_v[pl.ds(i * 16, 16)]
        m0 = d16 < HALF
        m1 = jnp.logical_not(m0)
        m0i = m0.astype(jnp.int32)
        m1i = m1.astype(jnp.int32)
        pos0 = off0 + plsc.cumsum(m0i) - 1
        plsc.store_scatter(os0_v, [pos0], s16, mask=m0)
        plsc.store_scatter(od0_v, [pos0], d16, mask=m0)
        pos1 = off1 + plsc.cumsum(m1i) - 1
        plsc.store_scatter(os1_v, [pos1], s16, mask=m1)
        plsc.store_scatter(od1_v, [pos1], d16 - HALF, mask=m1)
        return off0 + jnp.sum(m0i), off1 + jnp.sum(m1i)

    off0, off1 = lax.fori_loop(0, EPW // 16, rbody,
                               (jnp.int32(0), jnp.int32(0)))

    iota16 = lax.iota(jnp.int32, 16)
    for t, (off, os_v, od_v) in enumerate(
            ((off0, os0_v, od0_v), (off1, os1_v, od1_v))):
        npad = lax.rem(CH - lax.rem(off, CH), CH)
        pc = off + npad
        for k in range(CH // 16):
            lane = off + k * 16 + iota16
            mpad = lane < pc
            plsc.store_scatter(os_v, [lane], lane & TRASHM, mask=mpad)
            plsc.store_scatter(od_v, [lane], HALF + (lane & TRASHM), mask=mpad)
        base = (t * NW + wid) * CAP
        pltpu.sync_copy(os_v, rsrc_hbm.at[pl.ds(base, CAP)])
        pltpu.sync_copy(od_v, rdst_hbm.at[pl.ds(base, CAP)])
        cnt_v[...] = jnp.broadcast_to(pc, (16,))
        pltpu.sync_copy(cnt_v, cnt_hbm.at[pl.ds((t * NW + wid) * 16, 16)])


@functools.partial(
    pl.kernel,
    out_type=jax.ShapeDtypeStruct((NC, NPH, D), jnp.float32),
    mesh=_mesh,
    scratch_types=[
        [pltpu.VMEM((CH,), jnp.int32)] * 4,
        [pltpu.VMEM((CH,), jnp.int32)] * 4,
        [pltpu.VMEM((CH, D), jnp.float32)] * 4,
        pltpu.VMEM((16,), jnp.int32),
        pltpu.VMEM_SHARED((NPH, D), jnp.float32),
        [pltpu.SemaphoreType.DMA] * 4,
        [pltpu.SemaphoreType.DMA] * 4,
    ],
)
def _propagate_kernel(g_hbm, rsrc_hbm, rdst_hbm, cnt_hbm, zeros_hbm, out_hbm,
                      s_vs, d_vs, r_vs, cnt_v, acc_sh, sis, sgs):
    cid = lax.axis_index("c")
    sid = lax.axis_index("s")
    NB = 4
    pltpu.sync_copy(zeros_hbm, acc_sh.at[pl.ds(sid * RPS, RPS)])
    plsc.subcore_barrier()

    for li in range(2):
        lw = sid * 2 + li
        base = (cid * NW + lw) * CAP
        pltpu.sync_copy(cnt_hbm.at[pl.ds((cid * NW + lw) * 16, 16)], cnt_v)
        nch = lax.div(cnt_v[...][0], CH)

        def start_idx(c, b):
            pltpu.async_copy(rsrc_hbm.at[pl.ds(base + c * CH, CH)],
                             s_vs[b], sis[b])
            pltpu.async_copy(rdst_hbm.at[pl.ds(base + c * CH, CH)],
                             d_vs[b], sis[b])

        def wait_idx(b):
            pltpu.make_async_copy(rsrc_hbm.at[pl.ds(base, CH)],
                                  s_vs[b], sis[b]).wait()
            pltpu.make_async_copy(rdst_hbm.at[pl.ds(base, CH)],
                                  d_vs[b], sis[b]).wait()

        def start_gather(b):
            pltpu.async_copy(g_hbm.at[s_vs[b]], r_vs[b], sgs[b])

        def wait_gather(b):
            pltpu.make_async_copy(g_hbm.at[s_vs[b]], r_vs[b], sgs[b]).wait()

        def scatter(b):
            pltpu.sync_copy(r_vs[b], acc_sh.at[d_vs[b]], add=True)

        for p in range(NB - 1):
            @pl.when(p < nch)
            def _(p=p):
                start_idx(p, p)
                wait_idx(p)
                start_gather(p)

        @pl.when(NB - 1 < nch)
        def _():
            start_idx(NB - 1, NB - 1)

        def body(c, carry):
            for b in range(NB):
                @pl.when(lax.rem(c, NB) == b)
                def _(b=b):
                    bn = (b + NB - 1) % NB

                    @pl.when(c + NB - 1 < nch)
                    def _():
                        wait_idx(bn)
                        start_gather(bn)

                    wait_gather(b)
                    scatter(b)

                    @pl.when(c + NB < nch)
                    def _():
                        start_idx(c + NB, b)
            return carry

        lax.fori_loop(0, nch, body, 0)

    plsc.subcore_barrier()
    pltpu.sync_copy(acc_sh.at[pl.ds(sid * RPS, RPS)],
                    out_hbm.at[cid, pl.ds(sid * RPS, RPS)])


@functools.partial(
    pl.kernel,
    out_type=jax.ShapeDtypeStruct((NP2,), jnp.float32),
    mesh=_mesh,
    scratch_types=[
        pltpu.VMEM((NW, 1, MSL), jnp.float32),
        pltpu.VMEM((MSL,), jnp.float32),
    ],
)
def _merge_kernel(hist4_hbm, deg_hbm, blk_v, acc_v):
    cid = lax.axis_index("c")
    sid = lax.axis_index("s")
    wid = sid * NC + cid

    @pl.when(wid < NP2 // MSL)
    def _():
        pltpu.sync_copy(hist4_hbm.at[:, wid], blk_v)

        def vbody(v, carry):
            s = pl.ds(v * 16, 16)
            acc = blk_v[0, 0, s]
            for w in range(1, NW):
                acc = acc + blk_v[w, 0, s]
            acc_v[s] = acc
            return carry

        lax.fori_loop(0, MSL // 16, vbody, 0)
        pltpu.sync_copy(acc_v, deg_hbm.at[pl.ds(wid * MSL, MSL)])


BNM = 400


def _matmul_body(x_ref, w_ref, h_ref):
    h_ref[...] = jnp.dot(x_ref[...], w_ref[...],
                         preferred_element_type=jnp.float32)


_matmul_call = pl.pallas_call(
    _matmul_body,
    grid=(N // BNM,),
    in_specs=[
        pl.BlockSpec((BNM, D), lambda i: (i, 0)),
        pl.BlockSpec((D, D), lambda i: (0, 0)),
    ],
    out_specs=pl.BlockSpec((BNM, D), lambda i: (i, 0)),
    out_shape=jax.ShapeDtypeStruct((N, D), jnp.float32),
)


def _scale_body(h_ref, deg_ref, g_ref):
    dinv = lax.rsqrt(deg_ref[...] + 1.0)
    g_ref[...] = h_ref[...] * dinv


_scale_call = pl.pallas_call(
    _scale_body,
    grid=(N // BNM,),
    in_specs=[
        pl.BlockSpec((BNM, D), lambda i: (i, 0)),
        pl.BlockSpec((BNM, 1), lambda i: (i, 0)),
    ],
    out_specs=pl.BlockSpec((BNM, D), lambda i: (i, 0)),
    out_shape=jax.ShapeDtypeStruct((N, D), jnp.float32),
)

BNF = 200


def _finalize_body(accp_ref, g_ref, deg_ref, b_ref, out_ref):
    dinv = lax.rsqrt(deg_ref[...] + 1.0)
    out_ref[...] = (accp_ref[0] + g_ref[...]) * dinv + b_ref[...]


_finalize_call = pl.pallas_call(
    _finalize_body,
    grid=(N // BNF,),
    in_specs=[
        pl.BlockSpec((1, BNF, D), lambda i: (i // 25, i % 25, 0)),
        pl.BlockSpec((BNF, D), lambda i: (i, 0)),
        pl.BlockSpec((BNF, 1), lambda i: (i, 0)),
        pl.BlockSpec((1, D), lambda i: (0, 0)),
    ],
    out_specs=pl.BlockSpec((BNF, D), lambda i: (i, 0)),
    out_shape=jax.ShapeDtypeStruct((N, D), jnp.float32),
)


def kernel(x, edge_index, W, b):
    src = edge_index[0].reshape(E)
    dst = edge_index[1].reshape(E)
    zeros_acc = jnp.zeros((RPS, D), jnp.float32)
    h = _matmul_call(x, W)
    degp, rsrc, rdst, cnts = _route_kernel(src, dst)
    degm = _merge_kernel(degp.reshape(NW, NP2 // MSL, 1, MSL))
    deg = degm[:N].reshape(N, 1)
    g = _scale_call(h, deg)
    accp = _propagate_kernel(g, rsrc, rdst, cnts, zeros_acc)
    return _finalize_call(accp, g, deg, b.reshape(1, D))


# speedup vs baseline: 1.0483x; 1.0483x over previous
import functools

import jax
import jax.numpy as jnp
from jax import lax
from jax.experimental import pallas as pl
from jax.experimental.pallas import tpu as pltpu
from jax.experimental.pallas import tpu_sc as plsc

N = 10000
D = 128
E = 320000
NC = 2
NS = 16
NW = NC * NS
EPW = E // NW
CH = 128
HALF = N // 2
NPH = 5632
RPS = NPH // NS
ZB = 16
CAP = 10112
TRASHM = 511
NP2 = 10240
MSL = 512

_mesh = plsc.VectorSubcoreMesh(core_axis_name="c", subcore_axis_name="s",
                               num_cores=NC)


@functools.partial(
    pl.kernel,
    out_type=(
        jax.ShapeDtypeStruct((NW * NP2,), jnp.float32),
        jax.ShapeDtypeStruct((2 * NW * CAP,), jnp.int32),
        jax.ShapeDtypeStruct((2 * NW * CAP,), jnp.int32),
        jax.ShapeDtypeStruct((2 * NW * 16,), jnp.int32),
    ),
    mesh=_mesh,
    scratch_types=[
        pltpu.VMEM((EPW,), jnp.int32),
        pltpu.VMEM((EPW,), jnp.int32),
        pltpu.VMEM((NP2,), jnp.float32),
        pltpu.VMEM((CAP,), jnp.int32),
        pltpu.VMEM((CAP,), jnp.int32),
        pltpu.VMEM((CAP,), jnp.int32),
        pltpu.VMEM((CAP,), jnp.int32),
        pltpu.VMEM((16,), jnp.int32),
    ],
    compiler_params=pltpu.CompilerParams(needs_layout_passes=False),
)
def _route_kernel(src_hbm, dst_hbm, deg_hbm, rsrc_hbm, rdst_hbm, cnt_hbm,
                  src_v, dst_v, hist_v, os0_v, od0_v, os1_v, od1_v, cnt_v):
    cid = lax.axis_index("c")
    sid = lax.axis_index("s")
    wid = sid * NC + cid
    pltpu.sync_copy(src_hbm.at[pl.ds(wid * EPW, EPW)], src_v)
    pltpu.sync_copy(dst_hbm.at[pl.ds(wid * EPW, EPW)], dst_v)

    zeros16 = jnp.zeros((16,), jnp.float32)

    def zbody(i, carry):
        hist_v[pl.ds(i * 16, 16)] = zeros16
        return carry

    lax.fori_loop(0, NP2 // 16, zbody, 0)

    ones16 = jnp.ones((16,), jnp.float32)

    def hbody(i, carry):
        idx16 = dst_v[pl.ds(i * 16, 16)]
        plsc.addupdate_scatter(hist_v, [idx16], ones16)
        return carry

    lax.fori_loop(0, EPW // 16, hbody, 0)
    pltpu.sync_copy(hist_v, deg_hbm.at[pl.ds(wid * NP2, NP2)])

    def rbody(i, offs):
        off0, off1 = offs
        s16 = src_v[pl.ds(i * 16, 16)]
        d16 = dst_v[pl.ds(i * 16, 16)]
        m0 = d16 < HALF
        m1 = jnp.logical_not(m0)
        m0i = m0.astype(jnp.int32)
        m1i = m1.astype(jnp.int32)
        pos0 = off0 + plsc.cumsum(m0i) - 1
        plsc.store_scatter(os0_v, [pos0], s16, mask=m0)
        plsc.store_scatter(od0_v, [pos0], d16, mask=m0)
        pos1 = off1 + plsc.cumsum(m1i) - 1
        plsc.store_scatter(os1_v, [pos1], s16, mask=m1)
        plsc.store_scatter(od1_v, [pos1], d16 - HALF, mask=m1)
        return off0 + jnp.sum(m0i), off1 + jnp.sum(m1i)

    off0, off1 = lax.fori_loop(0, EPW // 16, rbody,
                               (jnp.int32(0), jnp.int32(0)))

    iota16 = lax.iota(jnp.int32, 16)
    for t, (off, os_v, od_v) in enumerate(
            ((off0, os0_v, od0_v), (off1, os1_v, od1_v))):
        npad = lax.rem(CH - lax.rem(off, CH), CH)
        pc = off + npad
        for k in range(CH // 16):
            lane = off + k * 16 + iota16
            mpad = lane < pc
            plsc.store_scatter(os_v, [lane], lane & TRASHM, mask=mpad)
            plsc.store_scatter(od_v, [lane], HALF + (lane & TRASHM), mask=mpad)
        base = (t * NW + wid) * CAP
        pltpu.sync_copy(os_v, rsrc_hbm.at[pl.ds(base, CAP)])
        pltpu.sync_copy(od_v, rdst_hbm.at[pl.ds(base, CAP)])
        cnt_v[...] = jnp.broadcast_to(pc, (16,))
        pltpu.sync_copy(cnt_v, cnt_hbm.at[pl.ds((t * NW + wid) * 16, 16)])


@functools.partial(
    pl.kernel,
    out_type=jax.ShapeDtypeStruct((NC, NPH, D), jnp.float32),
    mesh=_mesh,
    scratch_types=[
        [pltpu.VMEM((CH,), jnp.int32)] * 4,
        [pltpu.VMEM((CH,), jnp.int32)] * 4,
        [pltpu.VMEM((CH, D), jnp.float32)] * 4,
        pltpu.VMEM((16,), jnp.int32),
        pltpu.VMEM_SHARED((NPH, D), jnp.float32),
        [pltpu.SemaphoreType.DMA] * 4,
        [pltpu.SemaphoreType.DMA] * 4,
    ],
)
def _propagate_kernel(g_hbm, rsrc_hbm, rdst_hbm, cnt_hbm, zeros_hbm, out_hbm,
                      s_vs, d_vs, r_vs, cnt_v, acc_sh, sis, sgs):
    cid = lax.axis_index("c")
    sid = lax.axis_index("s")
    NB = 4
    pltpu.sync_copy(zeros_hbm, acc_sh.at[pl.ds(sid * RPS, RPS)])
    plsc.subcore_barrier()

    for li in range(2):
        lw = sid * 2 + li
        base = (cid * NW + lw) * CAP
        pltpu.sync_copy(cnt_hbm.at[pl.ds((cid * NW + lw) * 16, 16)], cnt_v)
        nch = lax.div(cnt_v[...][0], CH)

        def start_idx(c, b):
            pltpu.async_copy(rsrc_hbm.at[pl.ds(base + c * CH, CH)],
                             s_vs[b], sis[b])
            pltpu.async_copy(rdst_hbm.at[pl.ds(base + c * CH, CH)],
                             d_vs[b], sis[b])

        def wait_idx(b):
            pltpu.make_async_copy(rsrc_hbm.at[pl.ds(base, CH)],
                                  s_vs[b], sis[b]).wait()
            pltpu.make_async_copy(rdst_hbm.at[pl.ds(base, CH)],
                                  d_vs[b], sis[b]).wait()

        def start_gather(b):
            pltpu.async_copy(g_hbm.at[s_vs[b]], r_vs[b], sgs[b])

        def wait_gather(b):
            pltpu.make_async_copy(g_hbm.at[s_vs[b]], r_vs[b], sgs[b]).wait()

        def scatter(b):
            pltpu.sync_copy(r_vs[b], acc_sh.at[d_vs[b]], add=True)

        for p in range(NB - 1):
            @pl.when(p < nch)
            def _(p=p):
                start_idx(p, p)
                wait_idx(p)
                start_gather(p)

        @pl.when(NB - 1 < nch)
        def _():
            start_idx(NB - 1, NB - 1)

        def body(c, carry):
            for b in range(NB):
                @pl.when(lax.rem(c, NB) == b)
                def _(b=b):
                    bn = (b + NB - 1) % NB

                    @pl.when(c + NB - 1 < nch)
                    def _():
                        wait_idx(bn)
                        start_gather(bn)

                    wait_gather(b)
                    scatter(b)

                    @pl.when(c + NB < nch)
                    def _():
                        start_idx(c + NB, b)
            return carry

        lax.fori_loop(0, nch, body, 0)

    plsc.subcore_barrier()
    pltpu.sync_copy(acc_sh.at[pl.ds(sid * RPS, RPS)],
                    out_hbm.at[cid, pl.ds(sid * RPS, RPS)])


@functools.partial(
    pl.kernel,
    out_type=jax.ShapeDtypeStruct((NP2,), jnp.float32),
    mesh=_mesh,
    scratch_types=[
        pltpu.VMEM((NW, 1, MSL), jnp.float32),
        pltpu.VMEM((MSL,), jnp.float32),
    ],
)
def _merge_kernel(hist4_hbm, deg_hbm, blk_v, acc_v):
    cid = lax.axis_index("c")
    sid = lax.axis_index("s")
    wid = sid * NC + cid

    @pl.when(wid < NP2 // MSL)
    def _():
        pltpu.sync_copy(hist4_hbm.at[:, wid], blk_v)

        def vbody(v, carry):
            s = pl.ds(v * 16, 16)
            acc = blk_v[0, 0, s]
            for w in range(1, NW):
                acc = acc + blk_v[w, 0, s]
            acc_v[s] = acc
            return carry

        lax.fori_loop(0, MSL // 16, vbody, 0)
        pltpu.sync_copy(acc_v, deg_hbm.at[pl.ds(wid * MSL, MSL)])


BNM = 400


def _matmul_body(x_ref, w_ref, deg_ref, g_ref):
    dinv = lax.rsqrt(deg_ref[...] + 1.0)
    h = jnp.dot(x_ref[...], w_ref[...], preferred_element_type=jnp.float32)
    g_ref[...] = h * dinv


_matmul_call = pl.pallas_call(
    _matmul_body,
    grid=(N // BNM,),
    in_specs=[
        pl.BlockSpec((BNM, D), lambda i: (i, 0)),
        pl.BlockSpec((D, D), lambda i: (0, 0)),
        pl.BlockSpec((BNM, 1), lambda i: (i, 0)),
    ],
    out_specs=pl.BlockSpec((BNM, D), lambda i: (i, 0)),
    out_shape=jax.ShapeDtypeStruct((N, D), jnp.float32),
)

BNF = 200


def _finalize_body(accp_ref, g_ref, deg_ref, b_ref, out_ref):
    dinv = lax.rsqrt(deg_ref[...] + 1.0)
    out_ref[...] = (accp_ref[0] + g_ref[...]) * dinv + b_ref[...]


_finalize_call = pl.pallas_call(
    _finalize_body,
    grid=(N // BNF,),
    in_specs=[
        pl.BlockSpec((1, BNF, D), lambda i: (i // 25, i % 25, 0)),
        pl.BlockSpec((BNF, D), lambda i: (i, 0)),
        pl.BlockSpec((BNF, 1), lambda i: (i, 0)),
        pl.BlockSpec((1, D), lambda i: (0, 0)),
    ],
    out_specs=pl.BlockSpec((BNF, D), lambda i: (i, 0)),
    out_shape=jax.ShapeDtypeStruct((N, D), jnp.float32),
)


def kernel(x, edge_index, W, b):
    src = edge_index[0].reshape(E)
    dst = edge_index[1].reshape(E)
    zeros_acc = jnp.zeros((RPS, D), jnp.float32)
    degp, rsrc, rdst, cnts = _route_kernel(src, dst)
    degm = _merge_kernel(degp.reshape(NW, NP2 // MSL, 1, MSL))
    deg = degm[:N].reshape(N, 1)
    g = _matmul_call(x, W, deg)
    accp = _propagate_kernel(g, rsrc, rdst, cnts, zeros_acc)
    return _finalize_call(accp, g, deg, b.reshape(1, D))


# speedup vs baseline: 1.0601x; 1.0113x over previous
import functools

import jax
import jax.numpy as jnp
from jax import lax
from jax.experimental import pallas as pl
from jax.experimental.pallas import tpu as pltpu
from jax.experimental.pallas import tpu_sc as plsc

N = 10000
D = 128
E = 320000
NC = 2
NS = 16
NW = NC * NS
EPW = E // NW
CH = 128
HALF = N // 2
NPH = 5632
RPS = NPH // NS
ZB = 16
CAP = 10112
TRASHM = 511
NP2 = 10240
MSL = 512

_mesh = plsc.VectorSubcoreMesh(core_axis_name="c", subcore_axis_name="s",
                               num_cores=NC)


@functools.partial(
    pl.kernel,
    out_type=(
        jax.ShapeDtypeStruct((NC * NP2,), jnp.float32),
        jax.ShapeDtypeStruct((2 * NW * CAP,), jnp.int32),
        jax.ShapeDtypeStruct((2 * NW * CAP,), jnp.int32),
        jax.ShapeDtypeStruct((2 * NW * 16,), jnp.int32),
    ),
    mesh=_mesh,
    scratch_types=[
        pltpu.VMEM((EPW,), jnp.int32),
        pltpu.VMEM((EPW,), jnp.int32),
        pltpu.VMEM((NP2,), jnp.float32),
        pltpu.VMEM((CAP,), jnp.int32),
        pltpu.VMEM((CAP,), jnp.int32),
        pltpu.VMEM((CAP,), jnp.int32),
        pltpu.VMEM((CAP,), jnp.int32),
        pltpu.VMEM((16,), jnp.int32),
        pltpu.VMEM((NP2 // NS,), jnp.float32),
        pltpu.VMEM_SHARED((NS, 1, NP2), jnp.float32),
    ],
    compiler_params=pltpu.CompilerParams(needs_layout_passes=False),
)
def _route_kernel(src_hbm, dst_hbm, deg_hbm, rsrc_hbm, rdst_hbm, cnt_hbm,
                  src_v, dst_v, hist_v, os0_v, od0_v, os1_v, od1_v, cnt_v,
                  macc_v, stage_sh):
    cid = lax.axis_index("c")
    sid = lax.axis_index("s")
    wid = sid * NC + cid
    pltpu.sync_copy(src_hbm.at[pl.ds(wid * EPW, EPW)], src_v)
    pltpu.sync_copy(dst_hbm.at[pl.ds(wid * EPW, EPW)], dst_v)

    zeros16 = jnp.zeros((16,), jnp.float32)

    def zbody(i, carry):
        hist_v[pl.ds(i * 16, 16)] = zeros16
        return carry

    lax.fori_loop(0, NP2 // 16, zbody, 0)

    ones16 = jnp.ones((16,), jnp.float32)

    def hbody(i, carry):
        idx16 = dst_v[pl.ds(i * 16, 16)]
        plsc.addupdate_scatter(hist_v, [idx16], ones16)
        return carry

    lax.fori_loop(0, EPW // 16, hbody, 0)
    pltpu.sync_copy(hist_v, stage_sh.at[sid, 0])

    def rbody(i, offs):
        off0, off1 = offs
        s16 = src_v[pl.ds(i * 16, 16)]
        d16 = dst_v[pl.ds(i * 16, 16)]
        m0 = d16 < HALF
        m1 = jnp.logical_not(m0)
        m0i = m0.astype(jnp.int32)
        m1i = m1.astype(jnp.int32)
        pos0 = off0 + plsc.cumsum(m0i) - 1
        plsc.store_scatter(os0_v, [pos0], s16, mask=m0)
        plsc.store_scatter(od0_v, [pos0], d16, mask=m0)
        pos1 = off1 + plsc.cumsum(m1i) - 1
        plsc.store_scatter(os1_v, [pos1], s16, mask=m1)
        plsc.store_scatter(od1_v, [pos1], d16 - HALF, mask=m1)
        return off0 + jnp.sum(m0i), off1 + jnp.sum(m1i)

    off0, off1 = lax.fori_loop(0, EPW // 16, rbody,
                               (jnp.int32(0), jnp.int32(0)))

    iota16 = lax.iota(jnp.int32, 16)
    for t, (off, os_v, od_v) in enumerate(
            ((off0, os0_v, od0_v), (off1, os1_v, od1_v))):
        npad = lax.rem(CH - lax.rem(off, CH), CH)
        pc = off + npad
        for k in range(CH // 16):
            lane = off + k * 16 + iota16
            mpad = lane < pc
            plsc.store_scatter(os_v, [lane], lane & TRASHM, mask=mpad)
            plsc.store_scatter(od_v, [lane], HALF + (lane & TRASHM), mask=mpad)
        base = (t * NW + wid) * CAP
        pltpu.sync_copy(os_v, rsrc_hbm.at[pl.ds(base, CAP)])
        pltpu.sync_copy(od_v, rdst_hbm.at[pl.ds(base, CAP)])
        cnt_v[...] = jnp.broadcast_to(pc, (16,))
        pltpu.sync_copy(cnt_v, cnt_hbm.at[pl.ds((t * NW + wid) * 16, 16)])

    plsc.subcore_barrier()
    SLB = NP2 // NS
    for k in range(NS):
        pltpu.sync_copy(stage_sh.at[k, 0, pl.ds(sid * SLB, SLB)],
                        hist_v.at[pl.ds(k * SLB, SLB)])

    def mbody(v, carry):
        acc = hist_v[pl.ds(v * 16, 16)]
        for k in range(1, NS):
            acc = acc + hist_v[pl.ds(k * SLB + v * 16, 16)]
        macc_v[pl.ds(v * 16, 16)] = acc
        return carry

    lax.fori_loop(0, SLB // 16, mbody, 0)
    pltpu.sync_copy(macc_v, deg_hbm.at[pl.ds(cid * NP2 + sid * SLB, SLB)])


@functools.partial(
    pl.kernel,
    out_type=jax.ShapeDtypeStruct((NC, NPH, D), jnp.float32),
    mesh=_mesh,
    scratch_types=[
        [pltpu.VMEM((CH,), jnp.int32)] * 4,
        [pltpu.VMEM((CH,), jnp.int32)] * 4,
        [pltpu.VMEM((CH, D), jnp.float32)] * 4,
        pltpu.VMEM((16,), jnp.int32),
        pltpu.VMEM_SHARED((NPH, D), jnp.float32),
        [pltpu.SemaphoreType.DMA] * 4,
        [pltpu.SemaphoreType.DMA] * 4,
    ],
)
def _propagate_kernel(g_hbm, rsrc_hbm, rdst_hbm, cnt_hbm, zeros_hbm, out_hbm,
                      s_vs, d_vs, r_vs, cnt_v, acc_sh, sis, sgs):
    cid = lax.axis_index("c")
    sid = lax.axis_index("s")
    NB = 4
    pltpu.sync_copy(zeros_hbm, acc_sh.at[pl.ds(sid * RPS, RPS)])
    plsc.subcore_barrier()

    for li in range(2):
        lw = sid * 2 + li
        base = (cid * NW + lw) * CAP
        pltpu.sync_copy(cnt_hbm.at[pl.ds((cid * NW + lw) * 16, 16)], cnt_v)
        nch = lax.div(cnt_v[...][0], CH)

        def start_idx(c, b):
            pltpu.async_copy(rsrc_hbm.at[pl.ds(base + c * CH, CH)],
                             s_vs[b], sis[b])
            pltpu.async_copy(rdst_hbm.at[pl.ds(base + c * CH, CH)],
                             d_vs[b], sis[b])

        def wait_idx(b):
            pltpu.make_async_copy(rsrc_hbm.at[pl.ds(base, CH)],
                                  s_vs[b], sis[b]).wait()
            pltpu.make_async_copy(rdst_hbm.at[pl.ds(base, CH)],
                                  d_vs[b], sis[b]).wait()

        def start_gather(b):
            pltpu.async_copy(g_hbm.at[s_vs[b]], r_vs[b], sgs[b])

        def wait_gather(b):
            pltpu.make_async_copy(g_hbm.at[s_vs[b]], r_vs[b], sgs[b]).wait()

        def scatter(b):
            pltpu.sync_copy(r_vs[b], acc_sh.at[d_vs[b]], add=True)

        for p in range(NB - 1):
            @pl.when(p < nch)
            def _(p=p):
                start_idx(p, p)
                wait_idx(p)
                start_gather(p)

        @pl.when(NB - 1 < nch)
        def _():
            start_idx(NB - 1, NB - 1)

        def body(c, carry):
            for b in range(NB):
                @pl.when(lax.rem(c, NB) == b)
                def _(b=b):
                    bn = (b + NB - 1) % NB

                    @pl.when(c + NB - 1 < nch)
                    def _():
                        wait_idx(bn)
                        start_gather(bn)

                    wait_gather(b)
                    scatter(b)

                    @pl.when(c + NB < nch)
                    def _():
                        start_idx(c + NB, b)
            return carry

        lax.fori_loop(0, nch, body, 0)

    plsc.subcore_barrier()
    pltpu.sync_copy(acc_sh.at[pl.ds(sid * RPS, RPS)],
                    out_hbm.at[cid, pl.ds(sid * RPS, RPS)])


BNM = 400


def _matmul_body(x_ref, w_ref, deg_ref, g_ref):
    deg = deg_ref[:, 0:1] + deg_ref[:, 1:2] + 1.0
    dinv = lax.rsqrt(deg)
    h = jnp.dot(x_ref[...], w_ref[...], preferred_element_type=jnp.float32)
    g_ref[...] = h * dinv


_matmul_call = pl.pallas_call(
    _matmul_body,
    grid=(N // BNM,),
    in_specs=[
        pl.BlockSpec((BNM, D), lambda i: (i, 0)),
        pl.BlockSpec((D, D), lambda i: (0, 0)),
        pl.BlockSpec((BNM, 2), lambda i: (i, 0)),
    ],
    out_specs=pl.BlockSpec((BNM, D), lambda i: (i, 0)),
    out_shape=jax.ShapeDtypeStruct((N, D), jnp.float32),
)

BNF = 200


def _finalize_body(accp_ref, g_ref, deg_ref, b_ref, out_ref):
    deg = deg_ref[:, 0:1] + deg_ref[:, 1:2] + 1.0
    dinv = lax.rsqrt(deg)
    out_ref[...] = (accp_ref[0] + g_ref[...]) * dinv + b_ref[...]


_finalize_call = pl.pallas_call(
    _finalize_body,
    grid=(N // BNF,),
    in_specs=[
        pl.BlockSpec((1, BNF, D), lambda i: (i // 25, i % 25, 0)),
        pl.BlockSpec((BNF, D), lambda i: (i, 0)),
        pl.BlockSpec((BNF, 2), lambda i: (i, 0)),
        pl.BlockSpec((1, D), lambda i: (0, 0)),
    ],
    out_specs=pl.BlockSpec((BNF, D), lambda i: (i, 0)),
    out_shape=jax.ShapeDtypeStruct((N, D), jnp.float32),
)


def kernel(x, edge_index, W, b):
    src = edge_index[0].reshape(E)
    dst = edge_index[1].reshape(E)
    zeros_acc = jnp.zeros((RPS, D), jnp.float32)
    degp, rsrc, rdst, cnts = _route_kernel(src, dst)
    deg = degp.reshape(NC, NP2)[:, :N].T
    g = _matmul_call(x, W, deg)
    accp = _propagate_kernel(g, rsrc, rdst, cnts, zeros_acc)
    return _finalize_call(accp, g, deg, b.reshape(1, D))
